# baseline (speedup 1.0000x reference)
import functools

import jax
import jax.numpy as jnp
from jax import lax
from jax.experimental import pallas as pl
from jax.experimental.pallas import tpu as pltpu
from jax.experimental.pallas import tpu_sc as plsc

VOCAB = 100000
EMB = 128
HIDDEN = 1024
OUT = 1000
B = 4096
L = 5

_NUM_WORKERS = 32
_BPW = B // _NUM_WORKERS
_RPW = _BPW * L


def _sc_gather(table, idx_lmajor):
    mesh = plsc.VectorSubcoreMesh(core_axis_name="c", subcore_axis_name="s")

    @functools.partial(
        pl.kernel,
        mesh=mesh,
        out_type=jax.ShapeDtypeStruct((L * B, EMB), jnp.float32),
        scratch_types=[
            pltpu.VMEM((_RPW,), jnp.int32),
            pltpu.VMEM((_RPW, EMB), jnp.float32),
            pltpu.SemaphoreType.DMA,
        ],
    )
    def k(table_hbm, idx_hbm, out_hbm, idx_v, rows_v, sem):
        wid = lax.axis_index("s") * 2 + lax.axis_index("c")
        base = wid * _BPW
        for l in range(L):
            pltpu.sync_copy(idx_hbm.at[pl.ds(l * B + base, _BPW)],
                            idx_v.at[pl.ds(l * _BPW, _BPW)])
        copies = []
        for l in range(L):
            sl = pl.ds(l * _BPW, _BPW)
            copies.append(
                pltpu.async_copy(table_hbm.at[idx_v.at[sl]], rows_v.at[sl], sem)
            )
        for c in copies:
            c.wait()
        for l in range(L):
            pltpu.sync_copy(rows_v.at[pl.ds(l * _BPW, _BPW)],
                            out_hbm.at[pl.ds(l * B + base, _BPW)])

    return k(table, idx_lmajor)


def _mlp_body(x_ref, w1_ref, b1_ref, w2t_ref, b2t_ref, ot_ref):
    xs = x_ref[...].astype(jnp.bfloat16)
    xcat = jnp.concatenate([xs[l] for l in range(L)], axis=1)
    h = jnp.dot(xcat, w1_ref[...], preferred_element_type=jnp.float32)
    h = jnp.maximum(h + b1_ref[...], 0.0)
    ot = lax.dot_general(w2t_ref[...], h.astype(jnp.bfloat16),
                         (((1,), (1,)), ((), ())),
                         preferred_element_type=jnp.float32)
    ot_ref[...] = ot + b2t_ref[...]


_BM = 1024


def _tc_mlp(emb3, w1, b1, w2t, b2t):
    grid = (B // _BM,)
    return pl.pallas_call(
        _mlp_body,
        grid=grid,
        in_specs=[
            pl.BlockSpec((L, _BM, EMB), lambda i: (0, i, 0)),
            pl.BlockSpec((EMB * L, HIDDEN), lambda i: (0, 0)),
            pl.BlockSpec((1, HIDDEN), lambda i: (0, 0)),
            pl.BlockSpec((OUT, HIDDEN), lambda i: (0, 0)),
            pl.BlockSpec((OUT, 1), lambda i: (0, 0)),
        ],
        out_specs=pl.BlockSpec((OUT, _BM), lambda i: (0, i)),
        out_shape=jax.ShapeDtypeStruct((OUT, B), jnp.float32),
        compiler_params=pltpu.CompilerParams(
            dimension_semantics=("arbitrary",),
        ),
    )(emb3, w1, b1, w2t, b2t)


def kernel(x, emb_table, W1, b1, W2, b2):
    idx_lmajor = x.astype(jnp.int32).T.reshape(-1)
    rows = _sc_gather(emb_table, idx_lmajor)
    emb3 = rows.reshape(L, B, EMB)
    out_t = _tc_mlp(emb3, W1.astype(jnp.bfloat16), b1.reshape(1, HIDDEN),
                    W2.T.astype(jnp.bfloat16), b2.reshape(OUT, 1))
    return out_t.T


# speedup vs baseline: 1.0141x; 1.0141x over previous
import functools

import jax
import jax.numpy as jnp
from jax import lax
from jax.experimental import pallas as pl
from jax.experimental.pallas import tpu as pltpu
from jax.experimental.pallas import tpu_sc as plsc

VOCAB = 100000
EMB = 128
HIDDEN = 1024
OUT = 1000
B = 4096
L = 5

_NUM_WORKERS = 32
_BPW = B // _NUM_WORKERS
_RPW = _BPW * L


def _sc_gather(table, idx_lmajor):
    mesh = plsc.VectorSubcoreMesh(core_axis_name="c", subcore_axis_name="s")

    @functools.partial(
        pl.kernel,
        mesh=mesh,
        out_type=jax.ShapeDtypeStruct((L * B, EMB), jnp.float32),
        scratch_types=[
            pltpu.VMEM((_RPW,), jnp.int32),
            pltpu.VMEM((_RPW, EMB), jnp.float32),
            pltpu.SemaphoreType.DMA,
        ],
    )
    def k(table_hbm, idx_hbm, out_hbm, idx_v, rows_v, sem):
        wid = lax.axis_index("s") * 2 + lax.axis_index("c")
        base = wid * _BPW
        for l in range(L):
            pltpu.sync_copy(idx_hbm.at[pl.ds(l * B + base, _BPW)],
                            idx_v.at[pl.ds(l * _BPW, _BPW)])
        copies = []
        for l in range(L):
            sl = pl.ds(l * _BPW, _BPW)
            copies.append(
                pltpu.async_copy(table_hbm.at[idx_v.at[sl]], rows_v.at[sl], sem)
            )
        for c in copies:
            c.wait()
        for l in range(L):
            pltpu.sync_copy(rows_v.at[pl.ds(l * _BPW, _BPW)],
                            out_hbm.at[pl.ds(l * B + base, _BPW)])

    return k(table, idx_lmajor)


def _mlp_body(x_ref, w1_ref, b1_ref, w2t_ref, b2t_ref, ot_ref):
    xs = x_ref[...].astype(jnp.bfloat16)
    xcat = jnp.concatenate([xs[l] for l in range(L)], axis=1)
    h = jnp.dot(xcat, w1_ref[...], preferred_element_type=jnp.float32)
    h = jnp.maximum(h + b1_ref[...], 0.0)
    ot = lax.dot_general(w2t_ref[...], h.astype(jnp.bfloat16),
                         (((1,), (1,)), ((), ())),
                         preferred_element_type=jnp.float32)
    ot_ref[...] = ot + b2t_ref[...]


_BM = 512


def _tc_mlp(emb3, w1, b1, w2t, b2t):
    grid = (B // _BM,)
    return pl.pallas_call(
        _mlp_body,
        grid=grid,
        in_specs=[
            pl.BlockSpec((L, _BM, EMB), lambda i: (0, i, 0)),
            pl.BlockSpec((EMB * L, HIDDEN), lambda i: (0, 0)),
            pl.BlockSpec((1, HIDDEN), lambda i: (0, 0)),
            pl.BlockSpec((OUT, HIDDEN), lambda i: (0, 0)),
            pl.BlockSpec((OUT, 1), lambda i: (0, 0)),
        ],
        out_specs=pl.BlockSpec((OUT, _BM), lambda i: (0, i)),
        out_shape=jax.ShapeDtypeStruct((OUT, B), jnp.float32),
        compiler_params=pltpu.CompilerParams(
            dimension_semantics=("arbitrary",),
        ),
    )(emb3, w1, b1, w2t, b2t)


def kernel(x, emb_table, W1, b1, W2, b2):
    idx_lmajor = x.astype(jnp.int32).T.reshape(-1)
    rows = _sc_gather(emb_table, idx_lmajor)
    emb3 = rows.reshape(L, B, EMB)
    out_t = _tc_mlp(emb3, W1.astype(jnp.bfloat16), b1.reshape(1, HIDDEN),
                    W2.T.astype(jnp.bfloat16), b2.reshape(OUT, 1))
    return out_t.T


# speedup vs baseline: 1.0342x; 1.0198x over previous
import functools

import jax
import jax.numpy as jnp
from jax import lax
from jax.experimental import pallas as pl
from jax.experimental.pallas import tpu as pltpu
from jax.experimental.pallas import tpu_sc as plsc

VOCAB = 100000
EMB = 128
HIDDEN = 1024
OUT = 1000
B = 4096
L = 5

_NUM_WORKERS = 32
_BPW = B // _NUM_WORKERS
_RPW = _BPW * L


def _sc_gather(table, idx_lmajor):
    mesh = plsc.VectorSubcoreMesh(core_axis_name="c", subcore_axis_name="s")

    @functools.partial(
        pl.kernel,
        mesh=mesh,
        out_type=jax.ShapeDtypeStruct((L * B, EMB), jnp.float32),
        scratch_types=[
            pltpu.VMEM((_RPW,), jnp.int32),
            pltpu.VMEM((_RPW, EMB), jnp.float32),
            pltpu.SemaphoreType.DMA,
            pltpu.SemaphoreType.DMA,
            pltpu.SemaphoreType.DMA,
        ],
    )
    def k(table_hbm, idx_hbm, out_hbm, idx_v, rows_v, sem_i, sem_g, sem_o):
        wid = lax.axis_index("s") * 2 + lax.axis_index("c")
        base = wid * _BPW
        idx_c, gat_c, out_c = [], [], []
        for l in range(L):
            idx_c.append(pltpu.async_copy(
                idx_hbm.at[pl.ds(l * B + base, _BPW)],
                idx_v.at[pl.ds(l * _BPW, _BPW)], sem_i))
        for l in range(L):
            sl = pl.ds(l * _BPW, _BPW)
            idx_c[l].wait()
            gat_c.append(
                pltpu.async_copy(table_hbm.at[idx_v.at[sl]], rows_v.at[sl],
                                 sem_g))
        for l in range(L):
            sl = pl.ds(l * _BPW, _BPW)
            gat_c[l].wait()
            out_c.append(pltpu.async_copy(
                rows_v.at[sl], out_hbm.at[pl.ds(l * B + base, _BPW)], sem_o))
        for c in out_c:
            c.wait()

    return k(table, idx_lmajor)


def _mlp_body(x_ref, w1_ref, b1_ref, w2t_ref, b2t_ref, ot_ref):
    xs = x_ref[...].astype(jnp.bfloat16)
    xcat = jnp.concatenate([xs[l] for l in range(L)], axis=1)
    h = jnp.dot(xcat, w1_ref[...], preferred_element_type=jnp.float32)
    h = jnp.maximum(h + b1_ref[...], 0.0)
    ot = lax.dot_general(w2t_ref[...], h.astype(jnp.bfloat16),
                         (((1,), (1,)), ((), ())),
                         preferred_element_type=jnp.float32)
    ot_ref[...] = ot + b2t_ref[...]


_BM = 512


def _tc_mlp(emb3, w1, b1, w2t, b2t):
    grid = (B // _BM,)
    return pl.pallas_call(
        _mlp_body,
        grid=grid,
        in_specs=[
            pl.BlockSpec((L, _BM, EMB), lambda i: (0, i, 0)),
            pl.BlockSpec((EMB * L, HIDDEN), lambda i: (0, 0)),
            pl.BlockSpec((1, HIDDEN), lambda i: (0, 0)),
            pl.BlockSpec((OUT, HIDDEN), lambda i: (0, 0)),
            pl.BlockSpec((OUT, 1), lambda i: (0, 0)),
        ],
        out_specs=pl.BlockSpec((OUT, _BM), lambda i: (0, i)),
        out_shape=jax.ShapeDtypeStruct((OUT, B), jnp.float32),
        compiler_params=pltpu.CompilerParams(
            dimension_semantics=("arbitrary",),
        ),
    )(emb3, w1, b1, w2t, b2t)


def kernel(x, emb_table, W1, b1, W2, b2):
    idx_lmajor = x.astype(jnp.int32).T.reshape(-1)
    rows = _sc_gather(emb_table, idx_lmajor)
    emb3 = rows.reshape(L, B, EMB)
    out_t = _tc_mlp(emb3, W1.astype(jnp.bfloat16), b1.reshape(1, HIDDEN),
                    W2.T.astype(jnp.bfloat16), b2.reshape(OUT, 1))
    return out_t.T
